# Initial kernel scaffold; baseline (speedup 1.0000x reference)
import jax
import jax.numpy as jnp
from jax.experimental import pallas as pl


def kernel(seq, type_table, staff_table):
    raise NotImplementedError("write your pallas kernel here")


# speedup vs baseline: 5.5509x; 5.5509x over previous
import functools

import jax
import jax.numpy as jnp
from jax import lax
from jax.experimental import pallas as pl
from jax.experimental.pallas import tpu as pltpu
from jax.experimental.pallas import tpu_sc as plsc

D = 64
TMAX = 128
SMAX = 16
B, T = 4096, 200
TOK = B * T

NC, NS = 2, 16
NW = NC * NS
TPW = TOK // NW

GB = 128
IDX_ROWS = TOK // GB
ROWS_PER_W = IDX_ROWS // NW
IB = 4
BLK_TOK = IB * GB
NBLK = ROWS_PER_W // IB


def _prep_body(types_ref, staves_ref, tt_ref, st_ref, idx_ref, ct_ref):
    t = jnp.clip(types_ref[...], 0, TMAX - 1)
    s = jnp.clip(staves_ref[...], 0, SMAX - 1)
    idx_ref[...] = t * SMAX + s
    ct_ref[...] = tt_ref[...][:, None, :] + st_ref[...][None, :, :]


_prep = pl.pallas_call(
    _prep_body,
    out_shape=(
        jax.ShapeDtypeStruct((IDX_ROWS, GB), jnp.int32),
        jax.ShapeDtypeStruct((TMAX, SMAX, D), jnp.float32),
    ),
)


_sc_mesh = plsc.VectorSubcoreMesh(
    core_axis_name="c", subcore_axis_name="s", num_cores=NC, num_subcores=NS
)


@functools.partial(
    pl.kernel,
    out_type=jax.ShapeDtypeStruct((TOK, D), jnp.float32),
    scratch_types=[
        pltpu.VMEM((IB, GB), jnp.int32),
        pltpu.VMEM((BLK_TOK, D), jnp.float32),
        pltpu.SemaphoreType.DMA,
    ],
    mesh=_sc_mesh,
    compiler_params=pltpu.CompilerParams(use_tc_tiling_on_sc=False),
)
def _sc_gather(idx_hbm, ct_hbm, out_hbm, idx_v, rows_v, sem):
    wid = lax.axis_index("s") * NC + lax.axis_index("c")
    row0 = wid * ROWS_PER_W
    tok0 = wid * TPW

    def blk(b, carry):
        pltpu.sync_copy(idx_hbm.at[pl.ds(row0 + b * IB, IB)], idx_v)
        for j in range(IB):
            pltpu.async_copy(
                ct_hbm.at[idx_v.at[j]],
                rows_v.at[pl.ds(j * GB, GB)],
                sem,
            ).wait()
        pltpu.sync_copy(rows_v, out_hbm.at[pl.ds(tok0 + b * BLK_TOK, BLK_TOK)])
        return carry

    lax.fori_loop(0, NBLK, blk, 0)


def kernel(seq, type_table, staff_table):
    types = seq[:, :, 0].reshape(IDX_ROWS, GB)
    staves = seq[:, :, 1].reshape(IDX_ROWS, GB)
    idx, ct = _prep(types, staves, type_table, staff_table)
    out = _sc_gather(idx, ct.reshape(TMAX * SMAX, D))
    return out.reshape(B, T, D)
